# baseline (speedup 1.0000x reference)
import jax
import jax.numpy as jnp
from jax.experimental import pallas as pl
from jax.experimental.pallas import tpu as pltpu

N = 2048
BLK = 512
NBLK = N // BLK
L = 16
BIG = 1e30


def _chunk_items(s2, lane, lmod, lmod_f):
    lane_f = lane.astype(jnp.float32)
    y = (jnp.float32(N) - lane_f) - s2
    cs = y
    for d in (1, 2, 4, 8):
        cs = jnp.where(lmod >= d, cs + jnp.roll(cs, d, axis=1), cs)
    cs_excl = cs - y
    fit_y = jnp.full(s2.shape, -BIG, jnp.float32)
    for i_off in range(L):
        f = jnp.where(lmod == i_off, cs_excl, 0.0)
        for d in (1, 2, 4, 8):
            f = jnp.where(lmod >= d, f + jnp.roll(f, d, axis=1), f)
        len_f = lmod_f - jnp.float32(i_off) + 1.0
        mj = (cs - f) / len_f
        mj = jnp.where(lmod >= i_off, mj, BIG)
        for d in (1, 2, 4, 8):
            sh = jnp.where(lmod <= L - 1 - d, jnp.roll(mj, -d, axis=1), BIG)
            mj = jnp.minimum(mj, sh)
        fit_y = jnp.maximum(fit_y, jnp.where(lmod >= i_off, mj, -BIG))
    fit_z = -fit_y
    bnd = (lmod == 0) | (fit_z != jnp.roll(fit_z, 1, axis=1))
    t = jnp.where(bnd, lane, jnp.int32(1 << 20))
    sfx = t
    for d in (1, 2, 4, 8):
        sh = jnp.where(lmod <= L - 1 - d, jnp.roll(sfx, -d, axis=1), jnp.int32(1 << 20))
        sfx = jnp.minimum(sfx, sh)
    nxt_in = jnp.where(lmod <= L - 2, jnp.roll(sfx, -1, axis=1), jnp.int32(1 << 20))
    inext = jnp.minimum(nxt_in, lane - lmod + L)
    icnt = (inext - lane).astype(jnp.float32)
    isum = fit_z * icnt
    return isum, icnt, inext


def _rank_sort_body(
    vrows_ref, vcols_ref, pos_ref, s_ref, isum_ref, icnt_ref, inext_ref
):
    lane = jax.lax.broadcasted_iota(jnp.int32, (16, 128), 0) * 128 + \
        jax.lax.broadcasted_iota(jnp.int32, (16, 128), 1)
    lmod = lane & (L - 1)
    lmod_f = lmod.astype(jnp.float32)
    jj = jax.lax.broadcasted_iota(jnp.int32, (BLK, N), 1)
    ii0 = jax.lax.broadcasted_iota(jnp.int32, (BLK, N), 0)
    for r in range(2):
        vrow = vrows_ref[r : r + 1, :]

        ones_col = jnp.ones((N, 1), jnp.float32)

        def blk(b, acc):
            vi = vcols_ref[pl.ds(b * BLK, BLK), r : r + 1]
            before = (vrow > vi) | ((vrow == vi) & (jj < ii0 + b * BLK))
            bf = jnp.where(before, 1.0, 0.0)
            cnt = jax.lax.dot(bf, ones_col)
            pos_ref[pl.ds(b * BLK, BLK), r : r + 1] = cnt
            e = jnp.where(cnt.astype(jnp.int32) == jj, 1.0, 0.0)
            return acc + jax.lax.dot(vi.reshape(1, BLK), e)

        s_row = jax.lax.fori_loop(
            0, NBLK, blk, jnp.zeros((1, N), jnp.float32)
        )
        s_ref[r : r + 1, :] = s_row
        s2 = s_row.reshape(16, 128)
        isum, icnt, inext = _chunk_items(s2, lane, lmod, lmod_f)
        isum_ref[r, :, :] = isum
        icnt_ref[r, :, :] = icnt
        inext_ref[r, :, :] = inext


def _pav_loss_body(s_ref, pos_ref, isum_ref, icnt_ref, inext_ref, out_ref, means, counts):
    n = N
    fn = jnp.float32(N)
    lane_f = (
        jax.lax.broadcasted_iota(jnp.int32, (1, N), 1).astype(jnp.float32)
    )
    sols = []
    for r in range(2):

        def cond(carry):
            return carry[4]

        def trip(carry):
            k, sp, top_s, top_c, _ = carry
            prev = jnp.maximum(sp - 1, 0)
            ps = means[r, prev]
            pc = counts[r, prev]
            can_merge = (sp > 0) & (top_s * pc > ps * top_c)
            can_push = k < n
            ms = top_s + ps
            mc = top_c + pc
            kc = jnp.minimum(k, n - 1)
            push_s = isum_ref[r, kc]
            push_c = icnt_ref[r, kc]
            push_k = inext_ref[r, kc]
            st_idx = jnp.where(can_merge, prev, jnp.where(can_push, sp, n))
            means[r, st_idx] = jnp.where(can_merge, ms, top_s)
            counts[r, st_idx] = jnp.where(can_merge, mc, top_c)
            new_top_s = jnp.where(can_merge, ms, jnp.where(can_push, push_s, top_s))
            new_top_c = jnp.where(can_merge, mc, jnp.where(can_push, push_c, top_c))
            new_sp = sp + jnp.where(can_merge, -1, jnp.where(can_push, 1, 0))
            new_k = jnp.where(can_merge, k, jnp.where(can_push, push_k, k))
            nprev = jnp.maximum(new_sp - 1, 0)
            nps = means[r, nprev]
            npc = counts[r, nprev]
            nactive = ((new_sp > 0) & (new_top_s * npc > nps * new_top_c)) | (
                new_k < n
            )
            return new_k, new_sp, new_top_s, new_top_c, nactive

        k0 = inext_ref[r, 0]
        init = (k0, jnp.int32(0), isum_ref[r, 0], icnt_ref[r, 0], k0 < n)
        k, sp, top_s, top_c, _ = jax.lax.while_loop(cond, trip, init)
        means[r, sp] = top_s
        counts[r, sp] = top_c

        def fill(b, carry):
            start, dual = carry
            c = counts[r, b]
            m = means[r, b] / c
            dual = dual + jnp.where(
                (lane_f >= start) & (lane_f < start + c), m, 0.0
            )
            return start + c, dual

        _, dual = jax.lax.fori_loop(
            0, sp + 1, fill, (jnp.float32(0.0), jnp.zeros((1, N), jnp.float32))
        )
        sols.append(s_ref[r : r + 1, :] - dual)

    solp, solt = sols
    mp = jnp.sum(solp) / fn
    mt = jnp.sum(solt) / fn
    vp = jnp.sum((solp - mp) ** 2)
    vt = jnp.sum((solt - mt) ** 2)
    cp = solp - mp
    ct = solt - mt

    def dot_blk(b, acc):
        kk = jax.lax.broadcasted_iota(jnp.int32, (BLK, N), 1)
        pp = pos_ref[pl.ds(b * BLK, BLK), 0:1].astype(jnp.int32)
        pt = pos_ref[pl.ds(b * BLK, BLK), 1:2].astype(jnp.int32)
        rp = jnp.sum(jnp.where(pp == kk, cp, 0.0), axis=1, keepdims=True)
        rt = jnp.sum(jnp.where(pt == kk, ct, 0.0), axis=1, keepdims=True)
        return acc + jnp.sum(rp * rt)

    dot = jax.lax.fori_loop(0, NBLK, dot_blk, jnp.float32(0.0))
    out_ref[0, 0] = -dot / (jnp.sqrt(vp) * jnp.sqrt(vt))


def kernel(pred, target):
    vrows = jnp.concatenate([pred, target], axis=0)
    vcols = vrows.T
    pos, s, isum, icnt, inext = pl.pallas_call(
        _rank_sort_body,
        out_shape=(
            jax.ShapeDtypeStruct((N, 2), jnp.float32),
            jax.ShapeDtypeStruct((2, N), jnp.float32),
            jax.ShapeDtypeStruct((2, 16, 128), jnp.float32),
            jax.ShapeDtypeStruct((2, 16, 128), jnp.float32),
            jax.ShapeDtypeStruct((2, 16, 128), jnp.int32),
        ),
    )(vrows, vcols)
    isum = isum.reshape(2, N)
    icnt = icnt.reshape(2, N)
    inext = inext.reshape(2, N)
    loss = pl.pallas_call(
        _pav_loss_body,
        in_specs=[pl.BlockSpec(memory_space=pltpu.VMEM)] * 2
        + [pl.BlockSpec(memory_space=pltpu.SMEM)] * 3,
        out_specs=pl.BlockSpec(memory_space=pltpu.SMEM),
        out_shape=jax.ShapeDtypeStruct((1, 1), jnp.float32),
        scratch_shapes=[
            pltpu.SMEM((2, N + 1), jnp.float32),
            pltpu.SMEM((2, N + 1), jnp.float32),
        ],
    )(s, pos, isum, icnt, inext)
    return loss[0, 0]
